# baseline (speedup 1.0000x reference)
import jax
import jax.numpy as jnp
from jax import lax
from jax.experimental import pallas as pl
from jax.experimental.pallas import tpu as pltpu
from jax.experimental.pallas import tpu_sc as plsc

TOD, DOW, DOM, DOY = 288, 7, 31, 366
D = 64
B = 4096
S = 200
N_TOK = B * S
NC, NS = 2, 16
NW = NC * NS
ROWS_PER_W = B // NW
SC = 40
CPR = S // SC
N_CHUNK = ROWS_PER_W * CPR
R_DD = DOW * DOM
ROWS = TOD + R_DD + DOY


def _sc_body(x_hbm, tab_hbm, out_hbm, tab_v, x_v, out_v, sx, so):
    wid = lax.axis_index("s") * NC + lax.axis_index("c")
    pltpu.sync_copy(tab_hbm, tab_v)

    iota = lax.iota(jnp.int32, 16)
    lane = iota & 3
    quad = iota >> 2
    scale = jnp.where(lane == 0, float(TOD),
                      jnp.where(lane == 1, float(DOW),
                                jnp.where(lane == 2, float(DOM), float(DOY))))
    mul = jnp.where(lane == 1, DOM * D, D)
    off = jnp.where(lane == 1, TOD * D,
                    jnp.where(lane == 3, (TOD + R_DD) * D, 0))

    b0 = wid * ROWS_PER_W
    pltpu.async_copy(x_hbm.at[b0, pl.ds(0, SC)], x_v[0], sx[0])

    def compute_chunk(x_vp, out_vp):
        @plsc.parallel_loop(0, SC // 4, unroll=5)
        def quad_body(g):
            xv = plsc.load_gather(x_vp, [4 * g + quad, lane])
            cvec = ((xv + 0.5) * scale).astype(jnp.int32) * mul + off
            for k in range(4):
                r0 = cvec[4 * k]
                r1 = cvec[4 * k + 1] + cvec[4 * k + 2]
                r2 = cvec[4 * k + 3]
                t = 4 * g + k
                for c in range(0, D, 16):
                    v = (tab_v[pl.ds(r0 + c, 16)]
                         + tab_v[pl.ds(r1 + c, 16)]
                         + tab_v[pl.ds(r2 + c, 16)])
                    out_vp[t, pl.ds(c, 16)] = v

    def chunk_pair(cj, carry):
        for p in range(2):
            ci = 2 * cj + p
            b = b0 + ci // CPR
            s0 = (ci % CPR) * SC
            bn = b0 + (ci + 1) // CPR
            sn = ((ci + 1) % CPR) * SC
            bp = b0 + (ci - 2) // CPR
            sp = ((ci - 2) % CPR) * SC
            pltpu.make_async_copy(
                x_hbm.at[b, pl.ds(s0, SC)], x_v[p], sx[p]).wait()

            @pl.when(ci + 1 < N_CHUNK)
            def _():
                pltpu.async_copy(
                    x_hbm.at[bn, pl.ds(sn, SC)], x_v[1 - p], sx[1 - p])

            @pl.when(ci >= 2)
            def _():
                pltpu.make_async_copy(
                    out_v[p], out_hbm.at[bp, pl.ds(sp, SC)], so[p]).wait()

            compute_chunk(x_v[p], out_v[p])
            pltpu.async_copy(out_v[p], out_hbm.at[b, pl.ds(s0, SC)], so[p])
        return carry

    lax.fori_loop(0, N_CHUNK // 2, chunk_pair, 0)
    for p in range(2):
        ci = N_CHUNK - 2 + p
        b_last = b0 + ci // CPR
        s_last = (ci % CPR) * SC
        pltpu.make_async_copy(
            out_v[p], out_hbm.at[b_last, pl.ds(s_last, SC)], so[p]).wait()


def kernel(x, w_tod, w_dow, w_dom, w_doy):
    w_dd = (w_dow[:, None, :] + w_dom[None, :, :]).reshape(R_DD, D)
    tab = jnp.concatenate([w_tod, w_dd, w_doy], axis=0).reshape(-1)
    mesh = plsc.VectorSubcoreMesh(core_axis_name="c", subcore_axis_name="s")
    out = pl.kernel(
        _sc_body,
        out_type=jax.ShapeDtypeStruct((B, S, D), jnp.float32),
        mesh=mesh,
        scratch_types=[
            pltpu.VMEM((ROWS * D,), jnp.float32),
            [pltpu.VMEM((SC, 4), jnp.float32)] * 2,
            [pltpu.VMEM((SC, D), jnp.float32)] * 2,
            [pltpu.SemaphoreType.DMA] * 2,
            [pltpu.SemaphoreType.DMA] * 2,
        ],
        compiler_params=pltpu.CompilerParams(needs_layout_passes=False),
    )(x, tab)
    return out


# speedup vs baseline: 1.2640x; 1.2640x over previous
import jax
import jax.numpy as jnp
from jax import lax
from jax.experimental import pallas as pl
from jax.experimental.pallas import tpu as pltpu
from jax.experimental.pallas import tpu_sc as plsc

TOD, DOW, DOM, DOY = 288, 7, 31, 366
D = 64
B = 4096
S = 200
N_TOK = B * S
NC, NS = 2, 16
NW = NC * NS
ROWS_PER_W = B // NW
SIZES = (104, 96)
OFFS = (0, 104)
R_DD = DOW * DOM
ROWS = TOD + R_DD + DOY


def _sc_body(x_hbm, tab_hbm, out_hbm, tab_v, x_v, out_v, sx, so):
    wid = lax.axis_index("s") * NC + lax.axis_index("c")
    pltpu.sync_copy(tab_hbm, tab_v)

    iota = lax.iota(jnp.int32, 16)
    lane = iota & 3
    quad = iota >> 2
    scale = jnp.where(lane == 0, float(TOD),
                      jnp.where(lane == 1, float(DOW),
                                jnp.where(lane == 2, float(DOM), float(DOY))))
    mul = jnp.where(lane == 1, DOM * D, D)
    off = jnp.where(lane == 1, TOD * D,
                    jnp.where(lane == 3, (TOD + R_DD) * D, 0))

    b0 = wid * ROWS_PER_W
    pltpu.async_copy(x_hbm.at[b0, pl.ds(OFFS[0], SIZES[0])], x_v[0], sx[0])

    def compute_chunk(x_vp, out_vp, n_tok, unroll):
        @plsc.parallel_loop(0, n_tok // 4, unroll=unroll)
        def quad_body(g):
            xv = plsc.load_gather(x_vp, [4 * g + quad, lane])
            cvec = ((xv + 0.5) * scale).astype(jnp.int32) * mul + off
            for k in range(4):
                r0 = cvec[4 * k]
                r1 = cvec[4 * k + 1] + cvec[4 * k + 2]
                r2 = cvec[4 * k + 3]
                t = 4 * g + k
                for c in range(0, D, 16):
                    v = (tab_v[pl.ds(r0 + c, 16)]
                         + tab_v[pl.ds(r1 + c, 16)]
                         + tab_v[pl.ds(r2 + c, 16)])
                    out_vp[t, pl.ds(c, 16)] = v

    def row_body(cj, carry):
        b = b0 + cj
        for p in range(2):
            sz, o = SIZES[p], OFFS[p]
            pltpu.make_async_copy(
                x_hbm.at[b, pl.ds(o, sz)], x_v[p], sx[p]).wait()

            if p == 0:
                pltpu.async_copy(
                    x_hbm.at[b, pl.ds(OFFS[1], SIZES[1])], x_v[1], sx[1])
            else:
                @pl.when(cj + 1 < ROWS_PER_W)
                def _():
                    pltpu.async_copy(
                        x_hbm.at[b + 1, pl.ds(OFFS[0], SIZES[0])],
                        x_v[0], sx[0])

            @pl.when(cj >= 1)
            def _():
                pltpu.make_async_copy(
                    out_v[p], out_hbm.at[b - 1, pl.ds(o, sz)], so[p]).wait()

            compute_chunk(x_v[p], out_v[p], sz, 13 if p == 0 else 12)
            pltpu.async_copy(out_v[p], out_hbm.at[b, pl.ds(o, sz)], so[p])
        return carry

    lax.fori_loop(0, ROWS_PER_W, row_body, 0)
    b_last = b0 + ROWS_PER_W - 1
    for p in range(2):
        pltpu.make_async_copy(
            out_v[p], out_hbm.at[b_last, pl.ds(OFFS[p], SIZES[p])],
            so[p]).wait()


def kernel(x, w_tod, w_dow, w_dom, w_doy):
    w_dd = (w_dow[:, None, :] + w_dom[None, :, :]).reshape(R_DD, D)
    tab = jnp.concatenate([w_tod, w_dd, w_doy], axis=0).reshape(-1)
    mesh = plsc.VectorSubcoreMesh(core_axis_name="c", subcore_axis_name="s")
    out = pl.kernel(
        _sc_body,
        out_type=jax.ShapeDtypeStruct((B, S, D), jnp.float32),
        mesh=mesh,
        scratch_types=[
            pltpu.VMEM((ROWS * D,), jnp.float32),
            [pltpu.VMEM((SIZES[0], 4), jnp.float32),
             pltpu.VMEM((SIZES[1], 4), jnp.float32)],
            [pltpu.VMEM((SIZES[0], D), jnp.float32),
             pltpu.VMEM((SIZES[1], D), jnp.float32)],
            [pltpu.SemaphoreType.DMA] * 2,
            [pltpu.SemaphoreType.DMA] * 2,
        ],
        compiler_params=pltpu.CompilerParams(
            needs_layout_passes=False,
            internal_scratch_in_bytes=0,
        ),
    )(x, tab)
    return out
